# baseline (speedup 1.0000x reference)
import functools

import jax
import jax.numpy as jnp
from jax import lax
from jax.experimental import pallas as pl
from jax.experimental.pallas import tpu as pltpu
from jax.experimental.pallas import tpu_sc as plsc

_VOCAB = 100000
_EMBED = 64
_N_STEP = 20
_N_HIDDEN = 1024
_BATCH = 1024
_PAIR = 2 * _EMBED
_B_FLAT = _BATCH * _N_STEP
_NW = 32
_B_PER_W = _B_FLAT // _NW
_VB = 2560
_NB = (_VOCAB + _VB - 1) // _VB


def _sc_gather(table, idx):
    mesh = plsc.VectorSubcoreMesh(core_axis_name="c", subcore_axis_name="s")

    @functools.partial(
        pl.kernel,
        mesh=mesh,
        out_type=jax.ShapeDtypeStruct((_B_FLAT, _PAIR), jnp.float32),
        scratch_types=[
            pltpu.VMEM((_B_PER_W,), jnp.int32),
            pltpu.VMEM((_B_PER_W, _PAIR), jnp.float32),
            pltpu.SemaphoreType.DMA,
        ],
    )
    def gather_kernel(table_hbm, idx_hbm, out_hbm, idx_v, rows_v, sem):
        wid = lax.axis_index("s") * 2 + lax.axis_index("c")
        base = wid * _B_PER_W
        pltpu.sync_copy(idx_hbm.at[pl.ds(base, _B_PER_W)], idx_v)
        pltpu.async_copy(table_hbm.at[idx_v], rows_v, sem).wait()
        pltpu.sync_copy(rows_v, out_hbm.at[pl.ds(base, _B_PER_W)])

    return gather_kernel(table, idx)


def _h_kernel(raw_ref, w1_ref, b1_ref, h_ref):
    acc = jnp.zeros((_BATCH, _N_HIDDEN), jnp.float32)
    for t in range(_N_STEP):
        h0 = raw_ref[t].astype(jnp.bfloat16)
        acc += jnp.dot(h0, w1_ref[t], preferred_element_type=jnp.float32)
    h = jnp.tanh(acc + b1_ref[...])
    h_ref[...] = h.astype(jnp.bfloat16)


def _yt_kernel(h_ref, w2t_ref, b2_ref, yt_ref):
    acc = lax.dot_general(
        w2t_ref[...].astype(jnp.bfloat16), h_ref[...],
        dimension_numbers=(((1,), (1,)), ((), ())),
        preferred_element_type=jnp.float32)
    yt_ref[...] = acc + b2_ref[...].T


@jax.jit
def kernel(X, emb, W1, b1, W2, b2):
    idx_t = X.T.reshape(-1)
    table = jnp.zeros((_VOCAB, _PAIR), jnp.float32).at[:, :_EMBED].set(emb)
    raw = _sc_gather(table, idx_t)
    raw = raw.reshape(_N_STEP, _BATCH, _PAIR)
    w1_pad = jnp.pad(
        W1.reshape(_N_STEP, _EMBED, _N_HIDDEN),
        ((0, 0), (0, _PAIR - _EMBED), (0, 0))).astype(jnp.bfloat16)

    h = pl.pallas_call(
        _h_kernel,
        out_shape=jax.ShapeDtypeStruct((_BATCH, _N_HIDDEN), jnp.bfloat16),
    )(raw, w1_pad, b1.reshape(1, -1))

    yt = pl.pallas_call(
        _yt_kernel,
        grid=(_NB,),
        in_specs=[
            pl.BlockSpec((_BATCH, _N_HIDDEN), lambda j: (0, 0)),
            pl.BlockSpec((_VB, _N_HIDDEN), lambda j: (j, 0)),
            pl.BlockSpec((1, _VB), lambda j: (0, j)),
        ],
        out_specs=pl.BlockSpec((_VB, _N_HIDDEN), lambda j: (j, 0)),
        out_shape=jax.ShapeDtypeStruct((_VOCAB, _N_HIDDEN), jnp.float32),
    )(h, W2.T, b2.reshape(1, -1))
    return yt.T


# speedup vs baseline: 1.0070x; 1.0070x over previous
import functools

import jax
import jax.numpy as jnp
from jax import lax
from jax.experimental import pallas as pl
from jax.experimental.pallas import tpu as pltpu
from jax.experimental.pallas import tpu_sc as plsc

_VOCAB = 100000
_EMBED = 64
_N_STEP = 20
_N_HIDDEN = 1024
_BATCH = 1024
_PAIR = 2 * _EMBED
_B_FLAT = _BATCH * _N_STEP
_NW = 32
_B_PER_W = _B_FLAT // _NW
_VB = 2560
_NB = (_VOCAB + _VB - 1) // _VB


def _sc_gather(table, idx):
    mesh = plsc.VectorSubcoreMesh(core_axis_name="c", subcore_axis_name="s")

    @functools.partial(
        pl.kernel,
        mesh=mesh,
        out_type=jax.ShapeDtypeStruct((_B_FLAT, _PAIR), jnp.float32),
        scratch_types=[
            pltpu.VMEM((_B_PER_W,), jnp.int32),
            pltpu.VMEM((_B_PER_W, _PAIR), jnp.float32),
            pltpu.SemaphoreType.DMA,
        ],
    )
    def gather_kernel(table_hbm, idx_hbm, out_hbm, idx_v, rows_v, sem):
        wid = lax.axis_index("s") * 2 + lax.axis_index("c")
        base = wid * _B_PER_W
        pltpu.sync_copy(idx_hbm.at[pl.ds(base, _B_PER_W)], idx_v)
        pltpu.async_copy(table_hbm.at[idx_v], rows_v, sem).wait()
        pltpu.sync_copy(rows_v, out_hbm.at[pl.ds(base, _B_PER_W)])

    return gather_kernel(table, idx)


def _h_kernel(raw_ref, x_ref, w1_ref, b1_ref, h_ref):
    half = lax.broadcasted_iota(jnp.int32, (1, _PAIR), 1) // _EMBED
    acc = jnp.zeros((_BATCH, _N_HIDDEN), jnp.float32)
    for t in range(_N_STEP):
        par = x_ref[:, t:t + 1] & 1
        m = half == par
        h0m = jnp.where(m, raw_ref[t], 0.0).astype(jnp.bfloat16)
        acc += jnp.dot(h0m, w1_ref[t], preferred_element_type=jnp.float32)
    h = jnp.tanh(acc + b1_ref[...])
    h_ref[...] = h.astype(jnp.bfloat16)


def _yt_kernel(h_ref, w2t_ref, b2_ref, yt_ref):
    acc = lax.dot_general(
        w2t_ref[...].astype(jnp.bfloat16), h_ref[...],
        dimension_numbers=(((1,), (1,)), ((), ())),
        preferred_element_type=jnp.float32)
    yt_ref[...] = acc + b2_ref[...].T


@jax.jit
def kernel(X, emb, W1, b1, W2, b2):
    idx_t = X.T.reshape(-1)
    table = emb.T.reshape(_EMBED, _VOCAB // 2, 2).transpose(1, 2, 0)
    table = table.reshape(_VOCAB // 2, _PAIR)
    raw = _sc_gather(table, idx_t >> 1)
    raw = raw.reshape(_N_STEP, _BATCH, _PAIR)
    w1_exp = jnp.concatenate(
        [W1.reshape(_N_STEP, _EMBED, _N_HIDDEN)] * 2, axis=1
    ).astype(jnp.bfloat16)

    h = pl.pallas_call(
        _h_kernel,
        out_shape=jax.ShapeDtypeStruct((_BATCH, _N_HIDDEN), jnp.bfloat16),
    )(raw, X, w1_exp, b1.reshape(1, -1))

    yt = pl.pallas_call(
        _yt_kernel,
        grid=(_NB,),
        in_specs=[
            pl.BlockSpec((_BATCH, _N_HIDDEN), lambda j: (0, 0)),
            pl.BlockSpec((_VB, _N_HIDDEN), lambda j: (j, 0)),
            pl.BlockSpec((1, _VB), lambda j: (0, j)),
        ],
        out_specs=pl.BlockSpec((_VB, _N_HIDDEN), lambda j: (j, 0)),
        out_shape=jax.ShapeDtypeStruct((_VOCAB, _N_HIDDEN), jnp.float32),
    )(h, W2.T, b2.reshape(1, -1))
    return yt.T


# speedup vs baseline: 1.0723x; 1.0648x over previous
import functools

import jax
import jax.numpy as jnp
from jax import lax
from jax.experimental import pallas as pl
from jax.experimental.pallas import tpu as pltpu
from jax.experimental.pallas import tpu_sc as plsc

_VOCAB = 100000
_EMBED = 64
_N_STEP = 20
_N_HIDDEN = 1024
_BATCH = 1024
_PAIR = 2 * _EMBED
_B_FLAT = _BATCH * _N_STEP
_NW = 32
_B_PER_W = _B_FLAT // _NW
_VB = 2048
_NB = (_VOCAB + _VB - 1) // _VB


def _sc_gather(table, idx):
    mesh = plsc.VectorSubcoreMesh(core_axis_name="c", subcore_axis_name="s")

    @functools.partial(
        pl.kernel,
        mesh=mesh,
        out_type=jax.ShapeDtypeStruct((_B_FLAT, _PAIR), jnp.float32),
        scratch_types=[
            pltpu.VMEM((_B_PER_W,), jnp.int32),
            pltpu.VMEM((_B_PER_W, _PAIR), jnp.float32),
            pltpu.SemaphoreType.DMA,
        ],
    )
    def gather_kernel(table_hbm, idx_hbm, out_hbm, idx_v, rows_v, sem):
        wid = lax.axis_index("s") * 2 + lax.axis_index("c")
        base = wid * _B_PER_W
        pltpu.sync_copy(idx_hbm.at[pl.ds(base, _B_PER_W)], idx_v)
        pltpu.async_copy(table_hbm.at[idx_v], rows_v, sem).wait()
        pltpu.sync_copy(rows_v, out_hbm.at[pl.ds(base, _B_PER_W)])

    return gather_kernel(table, idx)


def _mlp_kernel(raw_ref, w1_ref, b1_ref, w2t_ref, b2_ref, yt_ref, h_ref):
    @pl.when(pl.program_id(0) == 0)
    def _():
        acc = jnp.zeros((_BATCH, _N_HIDDEN), jnp.float32)
        for t in range(_N_STEP):
            h0 = raw_ref[t]
            acc += jnp.dot(h0, w1_ref[t],
                           preferred_element_type=jnp.float32)
        h = jnp.tanh(acc + b1_ref[...])
        h_ref[...] = h.astype(jnp.bfloat16)

    acc = lax.dot_general(
        w2t_ref[...].astype(jnp.bfloat16), h_ref[...],
        dimension_numbers=(((1,), (1,)), ((), ())),
        preferred_element_type=jnp.float32)
    yt_ref[...] = acc + b2_ref[...].T


@jax.jit
def kernel(X, emb, W1, b1, W2, b2):
    idx_t = X.T.reshape(-1)
    table = jnp.pad(emb, ((0, 0), (0, _PAIR - _EMBED)))
    raw = _sc_gather(table, idx_t)
    raw = raw.reshape(_N_STEP, _BATCH, _PAIR).astype(jnp.bfloat16)
    w1_pad = jnp.pad(
        W1.reshape(_N_STEP, _EMBED, _N_HIDDEN),
        ((0, 0), (0, _PAIR - _EMBED), (0, 0))).astype(jnp.bfloat16)

    yt = pl.pallas_call(
        _mlp_kernel,
        grid=(_NB,),
        in_specs=[
            pl.BlockSpec((_N_STEP, _BATCH, _PAIR), lambda j: (0, 0, 0)),
            pl.BlockSpec((_N_STEP, _PAIR, _N_HIDDEN), lambda j: (0, 0, 0)),
            pl.BlockSpec((1, _N_HIDDEN), lambda j: (0, 0)),
            pl.BlockSpec((_VB, _N_HIDDEN), lambda j: (j, 0)),
            pl.BlockSpec((1, _VB), lambda j: (0, j)),
        ],
        out_specs=pl.BlockSpec((_VB, _N_HIDDEN), lambda j: (j, 0)),
        out_shape=jax.ShapeDtypeStruct((_VOCAB, _N_HIDDEN), jnp.float32),
        scratch_shapes=[pltpu.VMEM((_BATCH, _N_HIDDEN), jnp.bfloat16)],
    )(raw, w1_pad, b1.reshape(1, -1), W2.T, b2.reshape(1, -1))
    return yt.T


# speedup vs baseline: 1.1025x; 1.0281x over previous
import functools

import jax
import jax.numpy as jnp
from jax import lax
from jax.experimental import pallas as pl
from jax.experimental.pallas import tpu as pltpu
from jax.experimental.pallas import tpu_sc as plsc

_VOCAB = 100000
_EMBED = 64
_N_STEP = 20
_N_HIDDEN = 1024
_BATCH = 1024
_PAIR = 2 * _EMBED
_B_FLAT = _BATCH * _N_STEP
_NW = 32
_B_PER_W = _B_FLAT // _NW
_VB = 2816
_NB = (_VOCAB + _VB - 1) // _VB


def _sc_gather(table, idx):
    mesh = plsc.VectorSubcoreMesh(core_axis_name="c", subcore_axis_name="s")

    @functools.partial(
        pl.kernel,
        mesh=mesh,
        out_type=jax.ShapeDtypeStruct((_B_FLAT, _PAIR), jnp.float32),
        scratch_types=[
            pltpu.VMEM((_B_PER_W,), jnp.int32),
            pltpu.VMEM((_B_PER_W, _PAIR), jnp.float32),
            pltpu.SemaphoreType.DMA,
        ],
    )
    def gather_kernel(table_hbm, idx_hbm, out_hbm, idx_v, rows_v, sem):
        wid = lax.axis_index("s") * 2 + lax.axis_index("c")
        base = wid * _B_PER_W
        pltpu.sync_copy(idx_hbm.at[pl.ds(base, _B_PER_W)], idx_v)
        pltpu.async_copy(table_hbm.at[idx_v], rows_v, sem).wait()
        pltpu.sync_copy(rows_v, out_hbm.at[pl.ds(base, _B_PER_W)])

    return gather_kernel(table, idx)


def _h_kernel(raw_ref, w1_ref, b1_ref, h_ref):
    acc = jnp.zeros((_BATCH, _N_HIDDEN), jnp.float32)
    for t in range(_N_STEP):
        h0 = raw_ref[t].astype(jnp.bfloat16)
        acc += jnp.dot(h0, w1_ref[t], preferred_element_type=jnp.float32)
    h = jnp.tanh(acc + b1_ref[...])
    h_ref[...] = h.astype(jnp.bfloat16)


def _yt_kernel(h_ref, w2t_ref, b2_ref, yt_ref):
    acc = lax.dot_general(
        w2t_ref[...].astype(jnp.bfloat16), h_ref[...],
        dimension_numbers=(((1,), (1,)), ((), ())),
        preferred_element_type=jnp.float32)
    yt_ref[...] = acc + b2_ref[...].T


@jax.jit
def kernel(X, emb, W1, b1, W2, b2):
    idx_t = X.T.reshape(-1)
    table = jnp.pad(emb, ((0, 0), (0, _PAIR - _EMBED)))
    raw = _sc_gather(table, idx_t)
    raw = raw.reshape(_N_STEP, _BATCH, _PAIR)
    w1_pad = jnp.pad(
        W1.reshape(_N_STEP, _EMBED, _N_HIDDEN),
        ((0, 0), (0, _PAIR - _EMBED), (0, 0))).astype(jnp.bfloat16)

    h = pl.pallas_call(
        _h_kernel,
        out_shape=jax.ShapeDtypeStruct((_BATCH, _N_HIDDEN), jnp.bfloat16),
    )(raw, w1_pad, b1.reshape(1, -1))

    yt = pl.pallas_call(
        _yt_kernel,
        grid=(_NB,),
        in_specs=[
            pl.BlockSpec((_BATCH, _N_HIDDEN), lambda j: (0, 0)),
            pl.BlockSpec((_VB, _N_HIDDEN), lambda j: (j, 0)),
            pl.BlockSpec((1, _VB), lambda j: (0, j)),
        ],
        out_specs=pl.BlockSpec((_VB, _N_HIDDEN), lambda j: (j, 0)),
        out_shape=jax.ShapeDtypeStruct((_VOCAB, _N_HIDDEN), jnp.float32),
    )(h, W2.T, b2.reshape(1, -1))
    return yt.T
